# Initial kernel scaffold; baseline (speedup 1.0000x reference)
import jax
import jax.numpy as jnp
from jax import lax
from jax.experimental import pallas as pl
from jax.experimental.pallas import tpu as pltpu
from jax.experimental.pallas import tpu_sc as plsc

H = 128
NC = 2
NS = 16
NW = NC * NS

E = 80000
ACC1 = 12288
CH1 = 60
ACC2 = 2048
CH2 = 20


def _make_seg_sum(n_chunks, acc_rows, with_counts):
    mesh = plsc.VectorSubcoreMesh(core_axis_name="c", subcore_axis_name="s",
                                  num_cores=NC, num_subcores=NS)
    rows_per_tile = acc_rows // NS
    nz = rows_per_tile // 128

    def body(table, esrc, edst, *rest):
        if with_counts:
            (sums_out, cnt_out, src_v, dst_v, buf0, ones_v, zc_v,
             acc, cacc, sem0) = rest
        else:
            sums_out, src_v, dst_v, buf0, acc, sem0 = rest
        cid = lax.axis_index("c")
        sid = lax.axis_index("s")
        wid = cid * NS + sid

        zv = jnp.zeros((16,), jnp.float32)
        ov = jnp.ones((16,), jnp.float32)

        def fill_row(i, carry):
            for k in range(H // 16):
                buf0[i, k * 16:(k + 1) * 16] = zv
            if with_counts:
                ones_v[i, 0:16] = ov
                zc_v[i, 0:16] = zv
            return carry

        lax.fori_loop(0, 128, fill_row, 0)

        base = sid * rows_per_tile
        for j in range(nz):
            pltpu.sync_copy(buf0, acc.at[pl.ds(base + j * 128, 128)])
            if with_counts:
                pltpu.sync_copy(zc_v, cacc.at[pl.ds(base + j * 128, 128)])

        pltpu.sync_copy(esrc.at[wid], src_v)
        pltpu.sync_copy(edst.at[wid], dst_v)
        plsc.subcore_barrier()

        def step(j, carry):
            pltpu.async_copy(table.at[src_v.at[j]], buf0, sem0).wait()
            pltpu.sync_copy(buf0, acc.at[dst_v.at[j]], add=True)
            if with_counts:
                pltpu.sync_copy(ones_v, cacc.at[dst_v.at[j]], add=True)
            return carry

        lax.fori_loop(0, n_chunks, step, 0)
        plsc.subcore_barrier()

        for j in range(nz):
            r0 = base + j * 128
            pltpu.sync_copy(acc.at[pl.ds(r0, 128)],
                            sums_out.at[cid, pl.ds(r0, 128)])
            if with_counts:
                pltpu.sync_copy(cacc.at[pl.ds(r0, 128)],
                                cnt_out.at[cid, pl.ds(r0, 128)])

    out_type = [jax.ShapeDtypeStruct((NC, acc_rows, H), jnp.float32)]
    scratch = [
        pltpu.VMEM((n_chunks, 128), jnp.int32),
        pltpu.VMEM((n_chunks, 128), jnp.int32),
        pltpu.VMEM((128, H), jnp.float32),
    ]
    if with_counts:
        out_type.append(jax.ShapeDtypeStruct((NC, acc_rows, 16), jnp.float32))
        scratch += [pltpu.VMEM((128, 16), jnp.float32),
                    pltpu.VMEM((128, 16), jnp.float32)]
    scratch.append(pltpu.VMEM_SHARED((acc_rows, H), jnp.float32))
    if with_counts:
        scratch.append(pltpu.VMEM_SHARED((acc_rows, 16), jnp.float32))
    scratch.append(pltpu.SemaphoreType.DMA)

    return pl.kernel(body, out_type=tuple(out_type), mesh=mesh,
                     scratch_types=scratch)


_seg1 = _make_seg_sum(CH1, ACC1, True)
_seg2 = _make_seg_sum(CH2, ACC2, False)


def _proj_body(x_ref, w_ref, b_ref, o_ref):
    y = jnp.dot(x_ref[...], w_ref[0], preferred_element_type=jnp.float32)
    o_ref[...] = jnp.maximum(y + b_ref[0][None, :], 0.0)


def _input_proj(x_all, W_stack, b_stack):
    def sel(i):
        return jnp.where(i < 5, 0, jnp.where(i < 8, 1, 2))

    return pl.pallas_call(
        _proj_body,
        grid=(10,),
        in_specs=[pl.BlockSpec((1000, H), lambda i: (i, 0)),
                  pl.BlockSpec((1, H, H), lambda i: (sel(i), 0, 0)),
                  pl.BlockSpec((1, H), lambda i: (sel(i), 0))],
        out_specs=pl.BlockSpec((1000, H), lambda i: (i, 0)),
        out_shape=jax.ShapeDtypeStruct((10000, H), jnp.float32),
    )(x_all, W_stack, b_stack)


def _mean_from(s_ref, c_ref):
    cnt = jnp.sum(c_ref[0] + c_ref[1], axis=-1) * (1.0 / 16.0)
    return (s_ref[0] + s_ref[1]) / jnp.maximum(cnt, 1.0)[:, None]


def _ln_relu(x, g_ref, be_ref):
    mu = jnp.mean(x, axis=1, keepdims=True)
    var = jnp.mean((x - mu) * (x - mu), axis=1, keepdims=True)
    y = g_ref[...] * (x - mu) * lax.rsqrt(var + 1e-5) + be_ref[...]
    return jnp.maximum(y, 0.0)


def _update2_body(h_ref, sA_ref, cA_ref, sB_ref, cB_ref, wr_ref, wA_ref,
                  wB_ref, bc_ref, g_ref, be_ref, o_ref):
    x = jnp.dot(h_ref[...], wr_ref[...], preferred_element_type=jnp.float32)
    x = x + jnp.dot(_mean_from(sA_ref, cA_ref), wA_ref[...],
                    preferred_element_type=jnp.float32)
    x = x + jnp.dot(_mean_from(sB_ref, cB_ref), wB_ref[...],
                    preferred_element_type=jnp.float32)
    x = x + bc_ref[...]
    o_ref[...] = _ln_relu(x, g_ref, be_ref)


def _update1_body(h_ref, sA_ref, cA_ref, wr_ref, wA_ref, bc_ref, g_ref,
                  be_ref, o_ref):
    x = jnp.dot(h_ref[...], wr_ref[...], preferred_element_type=jnp.float32)
    x = x + jnp.dot(_mean_from(sA_ref, cA_ref), wA_ref[...],
                    preferred_element_type=jnp.float32)
    x = x + bc_ref[...]
    o_ref[...] = _ln_relu(x, g_ref, be_ref)


def _head_body(h_ref, sA_ref, cA_ref, wr_ref, wA_ref, bc_ref, g_ref,
               be_ref, wout_ref, bout_ref, base_ref, o_ref):
    x = jnp.dot(h_ref[...], wr_ref[...], preferred_element_type=jnp.float32)
    x = x + jnp.dot(_mean_from(sA_ref, cA_ref), wA_ref[...],
                    preferred_element_type=jnp.float32)
    x = x + bc_ref[...]
    h2 = _ln_relu(x, g_ref, be_ref)
    d = jnp.sum(h2 * wout_ref[...], axis=1, keepdims=True)
    o_ref[...] = jnp.broadcast_to(d + bout_ref[0, 0] + base_ref[0, 0],
                                  o_ref.shape)


def _row_spec(i):
    return (i, 0)


def _acc_spec(i):
    return (0, i, 0)


def _full2(shape):
    return pl.BlockSpec(shape, lambda i: (0,) * len(shape))


def _update2(h, sA, cA, sB, cB, wr, wA, wB, bc, g, be):
    n = h.shape[0]
    vec = lambda v: v.reshape(1, H)
    return pl.pallas_call(
        _update2_body,
        grid=(n // 1000,),
        in_specs=[pl.BlockSpec((1000, H), _row_spec),
                  pl.BlockSpec((2, 1000, H), _acc_spec),
                  pl.BlockSpec((2, 1000, 16), _acc_spec),
                  pl.BlockSpec((2, 1000, H), _acc_spec),
                  pl.BlockSpec((2, 1000, 16), _acc_spec),
                  _full2((H, H)), _full2((H, H)), _full2((H, H)),
                  _full2((1, H)), _full2((1, H)), _full2((1, H))],
        out_specs=pl.BlockSpec((1000, H), _row_spec),
        out_shape=jax.ShapeDtypeStruct((n, H), jnp.float32),
    )(h, sA, cA, sB, cB, wr, wA, wB, vec(bc), vec(g), vec(be))


def _update1(h, sA, cA, wr, wA, bc, g, be):
    n = h.shape[0]
    vec = lambda v: v.reshape(1, H)
    return pl.pallas_call(
        _update1_body,
        grid=(n // 1000,),
        in_specs=[pl.BlockSpec((1000, H), _row_spec),
                  pl.BlockSpec((2, 1000, H), _acc_spec),
                  pl.BlockSpec((2, 1000, 16), _acc_spec),
                  _full2((H, H)), _full2((H, H)),
                  _full2((1, H)), _full2((1, H)), _full2((1, H))],
        out_specs=pl.BlockSpec((1000, H), _row_spec),
        out_shape=jax.ShapeDtypeStruct((n, H), jnp.float32),
    )(h, sA, cA, wr, wA, vec(bc), vec(g), vec(be))


def _head(h, sA, cA, wr, wA, bc, g, be, wout, bout, base):
    n = h.shape[0]
    vec = lambda v: v.reshape(1, H)
    full = pl.pallas_call(
        _head_body,
        grid=(n // 1000,),
        in_specs=[pl.BlockSpec((1000, H), _row_spec),
                  pl.BlockSpec((2, 1000, H), _acc_spec),
                  pl.BlockSpec((2, 1000, 16), _acc_spec),
                  _full2((H, H)), _full2((H, H)),
                  _full2((1, H)), _full2((1, H)), _full2((1, H)),
                  _full2((1, H)), _full2((1, 1)), _full2((1, 1))],
        out_specs=pl.BlockSpec((1000, H), _row_spec),
        out_shape=jax.ShapeDtypeStruct((n, H), jnp.float32),
    )(h, sA, cA, wr, wA, vec(bc), vec(g), vec(be),
      wout.reshape(1, H), bout.reshape(1, 1), base.reshape(1, 1))
    return full[:, 0]


def kernel(x_papers, x_authors, x_assign, ei_r0, ei_r1, ei_r2, ei_r3,
           Win_papers, bin_papers, Win_authors, bin_authors, Win_assign,
           bin_assign, Wrel0, Wroot0, bconv0, g0, be0, Wrel1, Wroot1,
           bconv1, g1, be1, Wout, bout, base):
    x_all = jnp.concatenate([x_papers, x_authors, x_assign], axis=0)
    W_stack = jnp.stack([Win_papers, Win_authors, Win_assign])
    b_stack = jnp.stack([bin_papers, bin_authors, bin_assign])
    h0 = _input_proj(x_all, W_stack, b_stack)

    pad1 = NW * CH1 * 128 - 3 * E
    src1 = jnp.concatenate([ei_r1[0] + 5000, ei_r3[0] + 8000, ei_r2[0],
                            jnp.zeros((pad1,), jnp.int32)])
    dst1 = jnp.concatenate([ei_r1[1], ei_r3[1] + 5000, ei_r2[1] + 10000,
                            jnp.full((pad1,), 12000, jnp.int32)])
    sums1, cnt1 = _seg1(h0, src1.reshape(NW, CH1, 128),
                        dst1.reshape(NW, CH1, 128))

    h1p = _update2(h0[0:5000], sums1[:, 0:5000], cnt1[:, 0:5000],
                   sums1[:, 5000:10000], cnt1[:, 5000:10000],
                   Wroot0, Wrel0[1], Wrel0[3], bconv0, g0, be0)
    h1t = _update1(h0[8000:10000], sums1[:, 10000:12000],
                   cnt1[:, 10000:12000], Wroot0, Wrel0[2], bconv0, g0, be0)

    pad2 = NW * CH2 * 128 - E
    src2 = jnp.concatenate([ei_r2[0], jnp.zeros((pad2,), jnp.int32)])
    dst2 = jnp.concatenate([ei_r2[1], jnp.full((pad2,), 2000, jnp.int32)])
    (sums2,) = _seg2(h1p, src2.reshape(NW, CH2, 128),
                     dst2.reshape(NW, CH2, 128))

    return _head(h1t, sums2[:, 0:2000], cnt1[:, 10000:12000],
                 Wroot1, Wrel1[2], bconv1, g1, be1, Wout, bout, base)


# speedup vs baseline: 38.5561x; 38.5561x over previous
import jax
import jax.numpy as jnp
from jax import lax
from jax.experimental import pallas as pl
from jax.experimental.pallas import tpu as pltpu
from jax.experimental.pallas import tpu_sc as plsc

H = 128
NC = 2
NS = 16
NW = NC * NS

E = 80000
CH = 24
ACC_P = 5120
ACC_T = 2048


def _make_seg_sum(n_chunks, acc_rows, with_counts):
    mesh = plsc.VectorSubcoreMesh(core_axis_name="c", subcore_axis_name="s",
                                  num_cores=NC, num_subcores=NS)
    rows_per_tile = acc_rows // NS
    slabs = []
    off = 0
    while off < rows_per_tile:
        sz = min(128, rows_per_tile - off)
        slabs.append((off, sz))
        off += sz

    def body(table, esrc, edst, *rest):
        if with_counts:
            (sums_out, cnt_out, src_v, dst_v, buf0, ones_v, acc, cacc,
             sem0) = rest
        else:
            sums_out, src_v, dst_v, buf0, acc, sem0 = rest
        cid = lax.axis_index("c")
        sid = lax.axis_index("s")
        wid = cid * NS + sid

        zv = jnp.zeros((16,), jnp.float32)
        ov = jnp.ones((16,), jnp.float32)

        def fill_row(i, carry):
            for k in range(H // 16):
                buf0[i, k * 16:(k + 1) * 16] = zv
                if with_counts:
                    ones_v[i, k * 16:(k + 1) * 16] = ov
            return carry

        lax.fori_loop(0, 128, fill_row, 0)

        base = sid * rows_per_tile
        for off, sz in slabs:
            pltpu.sync_copy(buf0.at[pl.ds(0, sz)],
                            acc.at[pl.ds(base + off, sz)])
            if with_counts:
                pltpu.sync_copy(buf0.at[pl.ds(0, sz)],
                                cacc.at[pl.ds(base + off, sz)])

        pltpu.sync_copy(esrc.at[wid], src_v)
        pltpu.sync_copy(edst.at[wid], dst_v)
        plsc.subcore_barrier()

        def step(j, carry):
            pltpu.async_copy(table.at[src_v.at[j]], buf0, sem0).wait()
            pltpu.sync_copy(buf0, acc.at[dst_v.at[j]], add=True)
            if with_counts:
                pltpu.sync_copy(ones_v, cacc.at[dst_v.at[j]], add=True)
            return carry

        nch = jnp.clip((E - wid * (n_chunks * 128) + 127) // 128,
                       0, n_chunks)
        lax.fori_loop(0, nch, step, 0)
        plsc.subcore_barrier()

        for off, sz in slabs:
            r0 = base + off
            pltpu.sync_copy(acc.at[pl.ds(r0, sz)],
                            sums_out.at[cid, pl.ds(r0, sz)])
            if with_counts:
                pltpu.sync_copy(cacc.at[pl.ds(r0, sz)],
                                cnt_out.at[cid, pl.ds(r0, sz)])

    out_type = [jax.ShapeDtypeStruct((NC, acc_rows, H), jnp.float32)]
    scratch = [
        pltpu.VMEM((n_chunks, 128), jnp.int32),
        pltpu.VMEM((n_chunks, 128), jnp.int32),
        pltpu.VMEM((128, H), jnp.float32),
    ]
    if with_counts:
        out_type.append(jax.ShapeDtypeStruct((NC, acc_rows, H),
                                             jnp.float32))
        scratch.append(pltpu.VMEM((128, H), jnp.float32))
    scratch.append(pltpu.VMEM_SHARED((acc_rows, H), jnp.float32))
    if with_counts:
        scratch.append(pltpu.VMEM_SHARED((acc_rows, H), jnp.float32))
    scratch.append(pltpu.SemaphoreType.DMA)

    return pl.kernel(body, out_type=tuple(out_type), mesh=mesh,
                     scratch_types=scratch)


_seg_p = _make_seg_sum(CH, ACC_P, True)
_seg_t = _make_seg_sum(CH, ACC_T, True)
_seg_c = _make_seg_sum(CH, ACC_T, False)


def _edge_slices(srcs, dsts, n_chunks, dummy):
    total = NW * n_chunks * 128
    n = sum(s.shape[0] for s in srcs)
    pad = total - n
    src = jnp.concatenate(srcs + [jnp.zeros((pad,), jnp.int32)])
    dst = jnp.concatenate(dsts + [jnp.full((pad,), dummy, jnp.int32)])
    return src.reshape(NW, n_chunks, 128), dst.reshape(NW, n_chunks, 128)


def _proj_body(x_ref, w_ref, b_ref, o_ref):
    y = jnp.dot(x_ref[...], w_ref[0], preferred_element_type=jnp.float32)
    o_ref[...] = jnp.maximum(y + b_ref[0], 0.0)


def _input_proj(x_all, W_stack, b_stack):
    def sel(i):
        return jnp.where(i < 5, 0, jnp.where(i < 8, 1, 2))

    return pl.pallas_call(
        _proj_body,
        grid=(10,),
        in_specs=[pl.BlockSpec((1000, H), lambda i: (i, 0)),
                  pl.BlockSpec((1, H, H), lambda i: (sel(i), 0, 0)),
                  pl.BlockSpec((1, 1, H), lambda i: (sel(i), 0, 0))],
        out_specs=pl.BlockSpec((1000, H), lambda i: (i, 0)),
        out_shape=jax.ShapeDtypeStruct((10000, H), jnp.float32),
    )(x_all, W_stack, b_stack)


def _mean_from(s_ref, c_ref):
    return (s_ref[0] + s_ref[1]) / jnp.maximum(c_ref[0] + c_ref[1], 1.0)


def _ln_relu(x, g_ref, be_ref):
    mu = jnp.mean(x, axis=1, keepdims=True)
    var = jnp.mean((x - mu) * (x - mu), axis=1, keepdims=True)
    y = g_ref[...] * (x - mu) * lax.rsqrt(var + 1e-5) + be_ref[...]
    return jnp.maximum(y, 0.0)


def _update2_body(h_ref, sA_ref, cA_ref, sB_ref, cB_ref, wr_ref, wA_ref,
                  wB_ref, bc_ref, g_ref, be_ref, o_ref):
    x = jnp.dot(h_ref[...], wr_ref[...], preferred_element_type=jnp.float32)
    x = x + jnp.dot(_mean_from(sA_ref, cA_ref), wA_ref[...],
                    preferred_element_type=jnp.float32)
    x = x + jnp.dot(_mean_from(sB_ref, cB_ref), wB_ref[...],
                    preferred_element_type=jnp.float32)
    x = x + bc_ref[...]
    o_ref[...] = _ln_relu(x, g_ref, be_ref)


def _update1_body(h_ref, sA_ref, cA_ref, wr_ref, wA_ref, bc_ref, g_ref,
                  be_ref, o_ref):
    x = jnp.dot(h_ref[...], wr_ref[...], preferred_element_type=jnp.float32)
    x = x + jnp.dot(_mean_from(sA_ref, cA_ref), wA_ref[...],
                    preferred_element_type=jnp.float32)
    x = x + bc_ref[...]
    o_ref[...] = _ln_relu(x, g_ref, be_ref)


def _head_body(h_ref, sA_ref, cA_ref, wr_ref, wA_ref, bc_ref, g_ref,
               be_ref, wout_ref, bout_ref, base_ref, o_ref):
    x = jnp.dot(h_ref[...], wr_ref[...], preferred_element_type=jnp.float32)
    x = x + jnp.dot(_mean_from(sA_ref, cA_ref), wA_ref[...],
                    preferred_element_type=jnp.float32)
    x = x + bc_ref[...]
    h2 = _ln_relu(x, g_ref, be_ref)
    d = jnp.sum(h2 * wout_ref[...], axis=1, keepdims=True)
    o_ref[...] = jnp.broadcast_to(d + bout_ref[0, 0] + base_ref[0, 0],
                                  o_ref.shape)


def _row_spec(i):
    return (i, 0)


def _acc_spec(i):
    return (0, i, 0)


def _full2(shape):
    return pl.BlockSpec(shape, lambda i: (0,) * len(shape))


def _update2(h, sA, cA, sB, cB, wr, wA, wB, bc, g, be):
    n = h.shape[0]
    vec = lambda v: v.reshape(1, H)
    return pl.pallas_call(
        _update2_body,
        grid=(n // 1000,),
        in_specs=[pl.BlockSpec((1000, H), _row_spec),
                  pl.BlockSpec((2, 1000, H), _acc_spec),
                  pl.BlockSpec((2, 1000, H), _acc_spec),
                  pl.BlockSpec((2, 1000, H), _acc_spec),
                  pl.BlockSpec((2, 1000, H), _acc_spec),
                  _full2((H, H)), _full2((H, H)), _full2((H, H)),
                  _full2((1, H)), _full2((1, H)), _full2((1, H))],
        out_specs=pl.BlockSpec((1000, H), _row_spec),
        out_shape=jax.ShapeDtypeStruct((n, H), jnp.float32),
    )(h, sA, cA, sB, cB, wr, wA, wB, vec(bc), vec(g), vec(be))


def _update1(h, sA, cA, wr, wA, bc, g, be):
    n = h.shape[0]
    vec = lambda v: v.reshape(1, H)
    return pl.pallas_call(
        _update1_body,
        grid=(n // 1000,),
        in_specs=[pl.BlockSpec((1000, H), _row_spec),
                  pl.BlockSpec((2, 1000, H), _acc_spec),
                  pl.BlockSpec((2, 1000, H), _acc_spec),
                  _full2((H, H)), _full2((H, H)),
                  _full2((1, H)), _full2((1, H)), _full2((1, H))],
        out_specs=pl.BlockSpec((1000, H), _row_spec),
        out_shape=jax.ShapeDtypeStruct((n, H), jnp.float32),
    )(h, sA, cA, wr, wA, vec(bc), vec(g), vec(be))


def _head(h, sA, cA, wr, wA, bc, g, be, wout, bout, base):
    n = h.shape[0]
    vec = lambda v: v.reshape(1, H)
    full = pl.pallas_call(
        _head_body,
        grid=(n // 1000,),
        in_specs=[pl.BlockSpec((1000, H), _row_spec),
                  pl.BlockSpec((2, 1000, H), _acc_spec),
                  pl.BlockSpec((2, 1000, H), _acc_spec),
                  _full2((H, H)), _full2((H, H)),
                  _full2((1, H)), _full2((1, H)), _full2((1, H)),
                  _full2((1, H)), _full2((1, 1)), _full2((1, 1))],
        out_specs=pl.BlockSpec((1000, H), _row_spec),
        out_shape=jax.ShapeDtypeStruct((n, H), jnp.float32),
    )(h, sA, cA, wr, wA, vec(bc), vec(g), vec(be),
      wout.reshape(1, H), bout.reshape(1, 1), base.reshape(1, 1))
    return full[:, 0]


def kernel(x_papers, x_authors, x_assign, ei_r0, ei_r1, ei_r2, ei_r3,
           Win_papers, bin_papers, Win_authors, bin_authors, Win_assign,
           bin_assign, Wrel0, Wroot0, bconv0, g0, be0, Wrel1, Wroot1,
           bconv1, g1, be1, Wout, bout, base):
    x_all = jnp.concatenate([x_papers, x_authors, x_assign], axis=0)
    W_stack = jnp.stack([Win_papers, Win_authors, Win_assign])
    b_stack = jnp.stack([bin_papers, bin_authors, bin_assign]).reshape(3, 1, H)
    h0 = _input_proj(x_all, W_stack, b_stack)

    src1, dst1 = _edge_slices([ei_r1[0] + 5000], [ei_r1[1]], CH, 5000)
    sums1, cnt1 = _seg_p(h0, src1, dst1)
    src3, dst3 = _edge_slices([ei_r3[0] + 8000], [ei_r3[1]], CH, 5000)
    sums3, cnt3 = _seg_p(h0, src3, dst3)
    src2, dst2 = _edge_slices([ei_r2[0]], [ei_r2[1]], CH, 2000)
    sums2, cnt2 = _seg_t(h0, src2, dst2)

    h1p = _update2(h0[0:5000], sums1[:, 0:5000], cnt1[:, 0:5000],
                   sums3[:, 0:5000], cnt3[:, 0:5000],
                   Wroot0, Wrel0[1], Wrel0[3], bconv0, g0, be0)
    h1t = _update1(h0[8000:10000], sums2[:, 0:2000], cnt2[:, 0:2000],
                   Wroot0, Wrel0[2], bconv0, g0, be0)

    (sumsC,) = _seg_c(h1p, src2, dst2)

    return _head(h1t, sumsC[:, 0:2000], cnt2[:, 0:2000],
                 Wroot1, Wrel1[2], bconv1, g1, be1, Wout, bout, base)


# speedup vs baseline: 42.7803x; 1.1096x over previous
import jax
import jax.numpy as jnp
from jax import lax
from jax.experimental import pallas as pl
from jax.experimental.pallas import tpu as pltpu
from jax.experimental.pallas import tpu_sc as plsc

H = 128
NC = 2
NS = 16
NW = NC * NS

E = 80000
CH = 24
ACC_P = 5120
ACC_T = 2048
N_CHUNKS = E // 128
CH_BASE = N_CHUNKS // NW
CH_EXTRA = N_CHUNKS % NW


def _make_seg_sum(n_chunks, acc_rows, with_counts):
    mesh = plsc.VectorSubcoreMesh(core_axis_name="c", subcore_axis_name="s",
                                  num_cores=NC, num_subcores=NS)
    rows_per_tile = acc_rows // NS
    slabs = []
    off = 0
    while off < rows_per_tile:
        sz = min(128, rows_per_tile - off)
        slabs.append((off, sz))
        off += sz

    def body(table, esrc, edst, *rest):
        if with_counts:
            (sums_out, cnt_out, src_v, dst_v, buf0, ones_v, acc, cacc,
             sem0) = rest
        else:
            sums_out, src_v, dst_v, buf0, acc, sem0 = rest
        cid = lax.axis_index("c")
        sid = lax.axis_index("s")
        wid = cid * NS + sid

        zv = jnp.zeros((16,), jnp.float32)
        ov = jnp.ones((16,), jnp.float32)

        def fill_row(i, carry):
            for k in range(H // 16):
                buf0[i, k * 16:(k + 1) * 16] = zv
                if with_counts:
                    ones_v[i, k * 16:(k + 1) * 16] = ov
            return carry

        lax.fori_loop(0, 128, fill_row, 0)

        base = sid * rows_per_tile
        for off, sz in slabs:
            pltpu.sync_copy(buf0.at[pl.ds(0, sz)],
                            acc.at[pl.ds(base + off, sz)])
            if with_counts:
                pltpu.sync_copy(buf0.at[pl.ds(0, sz)],
                                cacc.at[pl.ds(base + off, sz)])

        pltpu.sync_copy(esrc.at[wid], src_v)
        pltpu.sync_copy(edst.at[wid], dst_v)
        plsc.subcore_barrier()

        def step(j, carry):
            pltpu.async_copy(table.at[src_v.at[j]], buf0, sem0).wait()
            pltpu.sync_copy(buf0, acc.at[dst_v.at[j]], add=True)
            if with_counts:
                pltpu.sync_copy(ones_v, cacc.at[dst_v.at[j]], add=True)
            return carry

        nch = jnp.where(wid < CH_EXTRA, CH_BASE + 1, CH_BASE)
        lax.fori_loop(0, nch, step, 0)
        plsc.subcore_barrier()

        for off, sz in slabs:
            r0 = base + off
            pltpu.sync_copy(acc.at[pl.ds(r0, sz)],
                            sums_out.at[cid, pl.ds(r0, sz)])
            if with_counts:
                pltpu.sync_copy(cacc.at[pl.ds(r0, sz)],
                                cnt_out.at[cid, pl.ds(r0, sz)])

    out_type = [jax.ShapeDtypeStruct((NC, acc_rows, H), jnp.float32)]
    scratch = [
        pltpu.VMEM((n_chunks, 128), jnp.int32),
        pltpu.VMEM((n_chunks, 128), jnp.int32),
        pltpu.VMEM((128, H), jnp.float32),
    ]
    if with_counts:
        out_type.append(jax.ShapeDtypeStruct((NC, acc_rows, H),
                                             jnp.float32))
        scratch.append(pltpu.VMEM((128, H), jnp.float32))
    scratch.append(pltpu.VMEM_SHARED((acc_rows, H), jnp.float32))
    if with_counts:
        scratch.append(pltpu.VMEM_SHARED((acc_rows, H), jnp.float32))
    scratch.append(pltpu.SemaphoreType.DMA)

    return pl.kernel(body, out_type=tuple(out_type), mesh=mesh,
                     scratch_types=scratch)


_seg_p = _make_seg_sum(CH, ACC_P, True)
_seg_t = _make_seg_sum(CH, ACC_T, True)
_seg_c = _make_seg_sum(CH, ACC_T, False)


def _chunk_map():
    import numpy as np
    m = np.full((NW, CH), N_CHUNKS, np.int32)
    start = 0
    for w in range(NW):
        n = CH_BASE + (1 if w < CH_EXTRA else 0)
        m[w, :n] = np.arange(start, start + n)
        start += n
    return m


_CHUNK_MAP = _chunk_map()


def _edge_slices(src, dst, dummy):
    src = jnp.concatenate([src.reshape(N_CHUNKS, 128),
                           jnp.zeros((1, 128), jnp.int32)])
    dst = jnp.concatenate([dst.reshape(N_CHUNKS, 128),
                           jnp.full((1, 128), dummy, jnp.int32)])
    return src[_CHUNK_MAP], dst[_CHUNK_MAP]


def _proj_body(x_ref, w_ref, b_ref, o_ref):
    y = jnp.dot(x_ref[...], w_ref[0], preferred_element_type=jnp.float32)
    o_ref[...] = jnp.maximum(y + b_ref[0], 0.0)


def _input_proj(x_all, W_stack, b_stack):
    def sel(i):
        return jnp.where(i < 5, 0, jnp.where(i < 8, 1, 2))

    return pl.pallas_call(
        _proj_body,
        grid=(10,),
        in_specs=[pl.BlockSpec((1000, H), lambda i: (i, 0)),
                  pl.BlockSpec((1, H, H), lambda i: (sel(i), 0, 0)),
                  pl.BlockSpec((1, 1, H), lambda i: (sel(i), 0, 0))],
        out_specs=pl.BlockSpec((1000, H), lambda i: (i, 0)),
        out_shape=jax.ShapeDtypeStruct((10000, H), jnp.float32),
    )(x_all, W_stack, b_stack)


def _mean_from(s_ref, c_ref):
    return (s_ref[0] + s_ref[1]) / jnp.maximum(c_ref[0] + c_ref[1], 1.0)


def _ln_relu(x, g_ref, be_ref):
    mu = jnp.mean(x, axis=1, keepdims=True)
    var = jnp.mean((x - mu) * (x - mu), axis=1, keepdims=True)
    y = g_ref[...] * (x - mu) * lax.rsqrt(var + 1e-5) + be_ref[...]
    return jnp.maximum(y, 0.0)


def _update2_body(h_ref, sA_ref, cA_ref, sB_ref, cB_ref, wr_ref, wA_ref,
                  wB_ref, bc_ref, g_ref, be_ref, o_ref):
    x = jnp.dot(h_ref[...], wr_ref[...], preferred_element_type=jnp.float32)
    x = x + jnp.dot(_mean_from(sA_ref, cA_ref), wA_ref[...],
                    preferred_element_type=jnp.float32)
    x = x + jnp.dot(_mean_from(sB_ref, cB_ref), wB_ref[...],
                    preferred_element_type=jnp.float32)
    x = x + bc_ref[...]
    o_ref[...] = _ln_relu(x, g_ref, be_ref)


def _update1_body(h_ref, sA_ref, cA_ref, wr_ref, wA_ref, bc_ref, g_ref,
                  be_ref, o_ref):
    x = jnp.dot(h_ref[...], wr_ref[...], preferred_element_type=jnp.float32)
    x = x + jnp.dot(_mean_from(sA_ref, cA_ref), wA_ref[...],
                    preferred_element_type=jnp.float32)
    x = x + bc_ref[...]
    o_ref[...] = _ln_relu(x, g_ref, be_ref)


def _head_body(h_ref, sA_ref, cA_ref, wr_ref, wA_ref, bc_ref, g_ref,
               be_ref, wout_ref, bout_ref, base_ref, o_ref):
    x = jnp.dot(h_ref[...], wr_ref[...], preferred_element_type=jnp.float32)
    x = x + jnp.dot(_mean_from(sA_ref, cA_ref), wA_ref[...],
                    preferred_element_type=jnp.float32)
    x = x + bc_ref[...]
    h2 = _ln_relu(x, g_ref, be_ref)
    d = jnp.sum(h2 * wout_ref[...], axis=1, keepdims=True)
    o_ref[...] = jnp.broadcast_to(d + bout_ref[0, 0] + base_ref[0, 0],
                                  o_ref.shape)


def _row_spec(i):
    return (i, 0)


def _acc_spec(i):
    return (0, i, 0)


def _full2(shape):
    return pl.BlockSpec(shape, lambda i: (0,) * len(shape))


def _update2(h, sA, cA, sB, cB, wr, wA, wB, bc, g, be):
    n = h.shape[0]
    vec = lambda v: v.reshape(1, H)
    return pl.pallas_call(
        _update2_body,
        grid=(n // 1000,),
        in_specs=[pl.BlockSpec((1000, H), _row_spec),
                  pl.BlockSpec((2, 1000, H), _acc_spec),
                  pl.BlockSpec((2, 1000, H), _acc_spec),
                  pl.BlockSpec((2, 1000, H), _acc_spec),
                  pl.BlockSpec((2, 1000, H), _acc_spec),
                  _full2((H, H)), _full2((H, H)), _full2((H, H)),
                  _full2((1, H)), _full2((1, H)), _full2((1, H))],
        out_specs=pl.BlockSpec((1000, H), _row_spec),
        out_shape=jax.ShapeDtypeStruct((n, H), jnp.float32),
    )(h, sA, cA, sB, cB, wr, wA, wB, vec(bc), vec(g), vec(be))


def _update1(h, sA, cA, wr, wA, bc, g, be):
    n = h.shape[0]
    vec = lambda v: v.reshape(1, H)
    return pl.pallas_call(
        _update1_body,
        grid=(n // 1000,),
        in_specs=[pl.BlockSpec((1000, H), _row_spec),
                  pl.BlockSpec((2, 1000, H), _acc_spec),
                  pl.BlockSpec((2, 1000, H), _acc_spec),
                  _full2((H, H)), _full2((H, H)),
                  _full2((1, H)), _full2((1, H)), _full2((1, H))],
        out_specs=pl.BlockSpec((1000, H), _row_spec),
        out_shape=jax.ShapeDtypeStruct((n, H), jnp.float32),
    )(h, sA, cA, wr, wA, vec(bc), vec(g), vec(be))


def _head(h, sA, cA, wr, wA, bc, g, be, wout, bout, base):
    n = h.shape[0]
    vec = lambda v: v.reshape(1, H)
    full = pl.pallas_call(
        _head_body,
        grid=(n // 1000,),
        in_specs=[pl.BlockSpec((1000, H), _row_spec),
                  pl.BlockSpec((2, 1000, H), _acc_spec),
                  pl.BlockSpec((2, 1000, H), _acc_spec),
                  _full2((H, H)), _full2((H, H)),
                  _full2((1, H)), _full2((1, H)), _full2((1, H)),
                  _full2((1, H)), _full2((1, 1)), _full2((1, 1))],
        out_specs=pl.BlockSpec((1000, H), _row_spec),
        out_shape=jax.ShapeDtypeStruct((n, H), jnp.float32),
    )(h, sA, cA, wr, wA, vec(bc), vec(g), vec(be),
      wout.reshape(1, H), bout.reshape(1, 1), base.reshape(1, 1))
    return full[:, 0]


def kernel(x_papers, x_authors, x_assign, ei_r0, ei_r1, ei_r2, ei_r3,
           Win_papers, bin_papers, Win_authors, bin_authors, Win_assign,
           bin_assign, Wrel0, Wroot0, bconv0, g0, be0, Wrel1, Wroot1,
           bconv1, g1, be1, Wout, bout, base):
    x_all = jnp.concatenate([x_papers, x_authors, x_assign], axis=0)
    W_stack = jnp.stack([Win_papers, Win_authors, Win_assign])
    b_stack = jnp.stack([bin_papers, bin_authors, bin_assign]).reshape(3, 1, H)
    h0 = _input_proj(x_all, W_stack, b_stack)

    src1, dst1 = _edge_slices(ei_r1[0] + 5000, ei_r1[1], 5000)
    sums1, cnt1 = _seg_p(h0, src1, dst1)
    src3, dst3 = _edge_slices(ei_r3[0] + 8000, ei_r3[1], 5000)
    sums3, cnt3 = _seg_p(h0, src3, dst3)
    src2, dst2 = _edge_slices(ei_r2[0], ei_r2[1], 2000)
    sums2, cnt2 = _seg_t(h0, src2, dst2)

    h1p = _update2(h0[0:5000], sums1[:, 0:5000], cnt1[:, 0:5000],
                   sums3[:, 0:5000], cnt3[:, 0:5000],
                   Wroot0, Wrel0[1], Wrel0[3], bconv0, g0, be0)
    h1t = _update1(h0[8000:10000], sums2[:, 0:2000], cnt2[:, 0:2000],
                   Wroot0, Wrel0[2], bconv0, g0, be0)

    (sumsC,) = _seg_c(h1p, src2, dst2)

    return _head(h1t, sumsC[:, 0:2000], cnt2[:, 0:2000],
                 Wroot1, Wrel1[2], bconv1, g1, be1, Wout, bout, base)


# speedup vs baseline: 48.2801x; 1.1286x over previous
import jax
import jax.numpy as jnp
from jax import lax
from jax.experimental import pallas as pl
from jax.experimental.pallas import tpu as pltpu
from jax.experimental.pallas import tpu_sc as plsc

H = 128
NC = 2
NS = 16
NW = NC * NS

E = 80000
CH = 24
ACC_P = 5120
ACC_T = 2048
N_CHUNKS = E // 128
CH_BASE = N_CHUNKS // NW
CH_EXTRA = N_CHUNKS % NW


def _make_seg_sum(n_chunks, acc_rows, with_counts):
    mesh = plsc.VectorSubcoreMesh(core_axis_name="c", subcore_axis_name="s",
                                  num_cores=NC, num_subcores=NS)
    rows_per_tile = acc_rows // NS
    slabs = []
    off = 0
    while off < rows_per_tile:
        sz = min(128, rows_per_tile - off)
        slabs.append((off, sz))
        off += sz

    def body(table, esrc, edst, *rest):
        if with_counts:
            (sums_out, cnt_out, src_v, dst_v, buf0, ones_v, acc, cacc,
             sem0) = rest
        else:
            sums_out, src_v, dst_v, buf0, acc, sem0 = rest
        cid = lax.axis_index("c")
        sid = lax.axis_index("s")
        wid = cid * NS + sid

        zv = jnp.zeros((16,), jnp.float32)
        ov = jnp.ones((16,), jnp.float32)

        def fill_row(i, carry):
            for k in range(H // 16):
                buf0[i, k * 16:(k + 1) * 16] = zv
                if with_counts:
                    ones_v[i, k * 16:(k + 1) * 16] = ov
            return carry

        lax.fori_loop(0, 128, fill_row, 0)

        base = sid * rows_per_tile
        for off, sz in slabs:
            pltpu.sync_copy(buf0.at[pl.ds(0, sz)],
                            acc.at[pl.ds(base + off, sz)])
            if with_counts:
                pltpu.sync_copy(buf0.at[pl.ds(0, sz)],
                                cacc.at[pl.ds(base + off, sz)])

        pltpu.sync_copy(esrc.at[wid], src_v)
        pltpu.sync_copy(edst.at[wid], dst_v)
        plsc.subcore_barrier()

        def step(j, carry):
            cp = pltpu.async_copy(table.at[src_v.at[j]], buf0, sem0)
            if with_counts:
                pltpu.sync_copy(ones_v, cacc.at[dst_v.at[j]], add=True)
            cp.wait()
            pltpu.sync_copy(buf0, acc.at[dst_v.at[j]], add=True)
            return carry

        nch = jnp.where(wid < CH_EXTRA, CH_BASE + 1, CH_BASE)
        lax.fori_loop(0, nch, step, 0)
        plsc.subcore_barrier()

        for off, sz in slabs:
            r0 = base + off
            pltpu.sync_copy(acc.at[pl.ds(r0, sz)],
                            sums_out.at[cid, pl.ds(r0, sz)])
            if with_counts:
                pltpu.sync_copy(cacc.at[pl.ds(r0, sz)],
                                cnt_out.at[cid, pl.ds(r0, sz)])

    out_type = [jax.ShapeDtypeStruct((NC, acc_rows, H), jnp.float32)]
    scratch = [
        pltpu.VMEM((n_chunks, 128), jnp.int32),
        pltpu.VMEM((n_chunks, 128), jnp.int32),
        pltpu.VMEM((128, H), jnp.float32),
    ]
    if with_counts:
        out_type.append(jax.ShapeDtypeStruct((NC, acc_rows, H),
                                             jnp.float32))
        scratch.append(pltpu.VMEM((128, H), jnp.float32))
    scratch.append(pltpu.VMEM_SHARED((acc_rows, H), jnp.float32))
    if with_counts:
        scratch.append(pltpu.VMEM_SHARED((acc_rows, H), jnp.float32))
    scratch.append(pltpu.SemaphoreType.DMA)

    return pl.kernel(body, out_type=tuple(out_type), mesh=mesh,
                     scratch_types=scratch)


_seg_p = _make_seg_sum(CH, ACC_P, True)
_seg_t = _make_seg_sum(CH, ACC_T, True)
_seg_c = _make_seg_sum(CH, ACC_T, False)


def _chunk_map():
    import numpy as np
    m = np.full((NW, CH), N_CHUNKS, np.int32)
    start = 0
    for w in range(NW):
        n = CH_BASE + (1 if w < CH_EXTRA else 0)
        m[w, :n] = np.arange(start, start + n)
        start += n
    return m


_CHUNK_MAP = _chunk_map()


def _edge_slices(src, dst, dummy):
    src = jnp.concatenate([src.reshape(N_CHUNKS, 128),
                           jnp.zeros((1, 128), jnp.int32)])
    dst = jnp.concatenate([dst.reshape(N_CHUNKS, 128),
                           jnp.full((1, 128), dummy, jnp.int32)])
    return src[_CHUNK_MAP], dst[_CHUNK_MAP]


def _proj_body(x_ref, w_ref, b_ref, o_ref):
    y = jnp.dot(x_ref[...], w_ref[0], preferred_element_type=jnp.float32)
    o_ref[...] = jnp.maximum(y + b_ref[0], 0.0)


def _input_proj(x_all, W_stack, b_stack):
    def sel(i):
        return jnp.where(i < 5, 0, jnp.where(i < 8, 1, 2))

    return pl.pallas_call(
        _proj_body,
        grid=(10,),
        in_specs=[pl.BlockSpec((1000, H), lambda i: (i, 0)),
                  pl.BlockSpec((1, H, H), lambda i: (sel(i), 0, 0)),
                  pl.BlockSpec((1, 1, H), lambda i: (sel(i), 0, 0))],
        out_specs=pl.BlockSpec((1000, H), lambda i: (i, 0)),
        out_shape=jax.ShapeDtypeStruct((10000, H), jnp.float32),
    )(x_all, W_stack, b_stack)


def _mean_from(s_ref, c_ref):
    return (s_ref[0] + s_ref[1]) / jnp.maximum(c_ref[0] + c_ref[1], 1.0)


def _ln_relu(x, g_ref, be_ref):
    mu = jnp.mean(x, axis=1, keepdims=True)
    var = jnp.mean((x - mu) * (x - mu), axis=1, keepdims=True)
    y = g_ref[...] * (x - mu) * lax.rsqrt(var + 1e-5) + be_ref[...]
    return jnp.maximum(y, 0.0)


def _update2_body(h_ref, sA_ref, cA_ref, sB_ref, cB_ref, wr_ref, wA_ref,
                  wB_ref, bc_ref, g_ref, be_ref, o_ref):
    x = jnp.dot(h_ref[...], wr_ref[...], preferred_element_type=jnp.float32)
    x = x + jnp.dot(_mean_from(sA_ref, cA_ref), wA_ref[...],
                    preferred_element_type=jnp.float32)
    x = x + jnp.dot(_mean_from(sB_ref, cB_ref), wB_ref[...],
                    preferred_element_type=jnp.float32)
    x = x + bc_ref[...]
    o_ref[...] = _ln_relu(x, g_ref, be_ref)


def _update1_body(h_ref, sA_ref, cA_ref, wr_ref, wA_ref, bc_ref, g_ref,
                  be_ref, o_ref):
    x = jnp.dot(h_ref[...], wr_ref[...], preferred_element_type=jnp.float32)
    x = x + jnp.dot(_mean_from(sA_ref, cA_ref), wA_ref[...],
                    preferred_element_type=jnp.float32)
    x = x + bc_ref[...]
    o_ref[...] = _ln_relu(x, g_ref, be_ref)


def _head_body(h_ref, sA_ref, cA_ref, wr_ref, wA_ref, bc_ref, g_ref,
               be_ref, wout_ref, bout_ref, base_ref, o_ref):
    x = jnp.dot(h_ref[...], wr_ref[...], preferred_element_type=jnp.float32)
    x = x + jnp.dot(_mean_from(sA_ref, cA_ref), wA_ref[...],
                    preferred_element_type=jnp.float32)
    x = x + bc_ref[...]
    h2 = _ln_relu(x, g_ref, be_ref)
    d = jnp.sum(h2 * wout_ref[...], axis=1, keepdims=True)
    o_ref[...] = jnp.broadcast_to(d + bout_ref[0, 0] + base_ref[0, 0],
                                  o_ref.shape)


def _row_spec(i):
    return (i, 0)


def _acc_spec(i):
    return (0, i, 0)


def _full2(shape):
    return pl.BlockSpec(shape, lambda i: (0,) * len(shape))


def _update2(h, sA, cA, sB, cB, wr, wA, wB, bc, g, be):
    n = h.shape[0]
    vec = lambda v: v.reshape(1, H)
    return pl.pallas_call(
        _update2_body,
        grid=(n // 1000,),
        in_specs=[pl.BlockSpec((1000, H), _row_spec),
                  pl.BlockSpec((2, 1000, H), _acc_spec),
                  pl.BlockSpec((2, 1000, H), _acc_spec),
                  pl.BlockSpec((2, 1000, H), _acc_spec),
                  pl.BlockSpec((2, 1000, H), _acc_spec),
                  _full2((H, H)), _full2((H, H)), _full2((H, H)),
                  _full2((1, H)), _full2((1, H)), _full2((1, H))],
        out_specs=pl.BlockSpec((1000, H), _row_spec),
        out_shape=jax.ShapeDtypeStruct((n, H), jnp.float32),
    )(h, sA, cA, sB, cB, wr, wA, wB, vec(bc), vec(g), vec(be))


def _update1(h, sA, cA, wr, wA, bc, g, be):
    n = h.shape[0]
    vec = lambda v: v.reshape(1, H)
    return pl.pallas_call(
        _update1_body,
        grid=(n // 1000,),
        in_specs=[pl.BlockSpec((1000, H), _row_spec),
                  pl.BlockSpec((2, 1000, H), _acc_spec),
                  pl.BlockSpec((2, 1000, H), _acc_spec),
                  _full2((H, H)), _full2((H, H)),
                  _full2((1, H)), _full2((1, H)), _full2((1, H))],
        out_specs=pl.BlockSpec((1000, H), _row_spec),
        out_shape=jax.ShapeDtypeStruct((n, H), jnp.float32),
    )(h, sA, cA, wr, wA, vec(bc), vec(g), vec(be))


def _head(h, sA, cA, wr, wA, bc, g, be, wout, bout, base):
    n = h.shape[0]
    vec = lambda v: v.reshape(1, H)
    full = pl.pallas_call(
        _head_body,
        grid=(n // 1000,),
        in_specs=[pl.BlockSpec((1000, H), _row_spec),
                  pl.BlockSpec((2, 1000, H), _acc_spec),
                  pl.BlockSpec((2, 1000, H), _acc_spec),
                  _full2((H, H)), _full2((H, H)),
                  _full2((1, H)), _full2((1, H)), _full2((1, H)),
                  _full2((1, H)), _full2((1, 1)), _full2((1, 1))],
        out_specs=pl.BlockSpec((1000, H), _row_spec),
        out_shape=jax.ShapeDtypeStruct((n, H), jnp.float32),
    )(h, sA, cA, wr, wA, vec(bc), vec(g), vec(be),
      wout.reshape(1, H), bout.reshape(1, 1), base.reshape(1, 1))
    return full[:, 0]


def kernel(x_papers, x_authors, x_assign, ei_r0, ei_r1, ei_r2, ei_r3,
           Win_papers, bin_papers, Win_authors, bin_authors, Win_assign,
           bin_assign, Wrel0, Wroot0, bconv0, g0, be0, Wrel1, Wroot1,
           bconv1, g1, be1, Wout, bout, base):
    x_all = jnp.concatenate([x_papers, x_authors, x_assign], axis=0)
    W_stack = jnp.stack([Win_papers, Win_authors, Win_assign])
    b_stack = jnp.stack([bin_papers, bin_authors, bin_assign]).reshape(3, 1, H)
    h0 = _input_proj(x_all, W_stack, b_stack)

    src1, dst1 = _edge_slices(ei_r1[0] + 5000, ei_r1[1], 5000)
    sums1, cnt1 = _seg_p(h0, src1, dst1)
    src3, dst3 = _edge_slices(ei_r3[0] + 8000, ei_r3[1], 5000)
    sums3, cnt3 = _seg_p(h0, src3, dst3)
    src2, dst2 = _edge_slices(ei_r2[0], ei_r2[1], 2000)
    sums2, cnt2 = _seg_t(h0, src2, dst2)

    h1p = _update2(h0[0:5000], sums1[:, 0:5000], cnt1[:, 0:5000],
                   sums3[:, 0:5000], cnt3[:, 0:5000],
                   Wroot0, Wrel0[1], Wrel0[3], bconv0, g0, be0)
    h1t = _update1(h0[8000:10000], sums2[:, 0:2000], cnt2[:, 0:2000],
                   Wroot0, Wrel0[2], bconv0, g0, be0)

    (sumsC,) = _seg_c(h1p, src2, dst2)

    return _head(h1t, sumsC[:, 0:2000], cnt2[:, 0:2000],
                 Wroot1, Wrel1[2], bconv1, g1, be1, Wout, bout, base)


# speedup vs baseline: 51.3853x; 1.0643x over previous
import jax
import jax.numpy as jnp
from jax import lax
from jax.experimental import pallas as pl
from jax.experimental.pallas import tpu as pltpu
from jax.experimental.pallas import tpu_sc as plsc

H = 128
NC = 2
NS = 16
NW = NC * NS

E = 80000
CH = 24
ACC_P = 5120
ACC_T = 2048
N_CHUNKS = E // 128
CH_BASE = N_CHUNKS // NW
CH_EXTRA = N_CHUNKS % NW


def _make_seg_sum(n_chunks, acc_rows, with_counts):
    mesh = plsc.VectorSubcoreMesh(core_axis_name="c", subcore_axis_name="s",
                                  num_cores=NC, num_subcores=NS)
    rows_per_tile = acc_rows // NS
    slabs = []
    off = 0
    while off < rows_per_tile:
        sz = min(128, rows_per_tile - off)
        slabs.append((off, sz))
        off += sz

    def body(table, esrc, edst, *rest):
        if with_counts:
            (sums_out, cnt_out, src_v, dst_v, buf0, ones_v, acc, cacc,
             sem0) = rest
        else:
            sums_out, src_v, dst_v, buf0, buf1, acc, sem0, sem1 = rest
        cid = lax.axis_index("c")
        sid = lax.axis_index("s")
        wid = cid * NS + sid

        zv = jnp.zeros((16,), jnp.float32)
        ov = jnp.ones((16,), jnp.float32)

        def fill_row(i, carry):
            for k in range(H // 16):
                buf0[i, k * 16:(k + 1) * 16] = zv
                if with_counts:
                    ones_v[i, k * 16:(k + 1) * 16] = ov
            return carry

        lax.fori_loop(0, 128, fill_row, 0)

        base = sid * rows_per_tile
        for off, sz in slabs:
            pltpu.sync_copy(buf0.at[pl.ds(0, sz)],
                            acc.at[pl.ds(base + off, sz)])
            if with_counts:
                pltpu.sync_copy(buf0.at[pl.ds(0, sz)],
                                cacc.at[pl.ds(base + off, sz)])

        pltpu.sync_copy(esrc.at[wid], src_v)
        pltpu.sync_copy(edst.at[wid], dst_v)
        plsc.subcore_barrier()

        nch = jnp.where(wid < CH_EXTRA, CH_BASE + 1, CH_BASE)

        if with_counts:
            def step(j, carry):
                cp = pltpu.async_copy(table.at[src_v.at[j]], buf0, sem0)
                pltpu.sync_copy(ones_v, cacc.at[dst_v.at[j]], add=True)
                cp.wait()
                pltpu.sync_copy(buf0, acc.at[dst_v.at[j]], add=True)
                return carry

            lax.fori_loop(0, nch, step, 0)
        else:
            def step2(i, carry):
                j0 = 2 * i
                j1 = j0 + 1

                @pl.when(j1 < nch)
                def _():
                    pltpu.async_copy(table.at[src_v.at[j1]], buf1, sem1)

                pltpu.make_async_copy(table.at[src_v.at[j0]], buf0,
                                      sem0).wait()
                pltpu.sync_copy(buf0, acc.at[dst_v.at[j0]], add=True)

                @pl.when(j0 + 2 < nch)
                def _():
                    pltpu.async_copy(table.at[src_v.at[j0 + 2]], buf0,
                                     sem0)

                @pl.when(j1 < nch)
                def _():
                    pltpu.make_async_copy(table.at[src_v.at[j1]], buf1,
                                          sem1).wait()
                    pltpu.sync_copy(buf1, acc.at[dst_v.at[j1]], add=True)

                return carry

            pltpu.async_copy(table.at[src_v.at[0]], buf0, sem0)
            lax.fori_loop(0, (CH_BASE + 2) // 2, step2, 0)
        plsc.subcore_barrier()

        for off, sz in slabs:
            r0 = base + off
            pltpu.sync_copy(acc.at[pl.ds(r0, sz)],
                            sums_out.at[cid, pl.ds(r0, sz)])
            if with_counts:
                pltpu.sync_copy(cacc.at[pl.ds(r0, sz)],
                                cnt_out.at[cid, pl.ds(r0, sz)])

    out_type = [jax.ShapeDtypeStruct((NC, acc_rows, H), jnp.float32)]
    scratch = [
        pltpu.VMEM((n_chunks, 128), jnp.int32),
        pltpu.VMEM((n_chunks, 128), jnp.int32),
        pltpu.VMEM((128, H), jnp.float32),
    ]
    if with_counts:
        out_type.append(jax.ShapeDtypeStruct((NC, acc_rows, H),
                                             jnp.float32))
        scratch.append(pltpu.VMEM((128, H), jnp.float32))
    else:
        scratch.append(pltpu.VMEM((128, H), jnp.float32))
    scratch.append(pltpu.VMEM_SHARED((acc_rows, H), jnp.float32))
    if with_counts:
        scratch.append(pltpu.VMEM_SHARED((acc_rows, H), jnp.float32))
    scratch.append(pltpu.SemaphoreType.DMA)
    if not with_counts:
        scratch.append(pltpu.SemaphoreType.DMA)

    return pl.kernel(body, out_type=tuple(out_type), mesh=mesh,
                     scratch_types=scratch)


_seg_p = _make_seg_sum(CH, ACC_P, True)
_seg_t = _make_seg_sum(CH, ACC_T, True)
_seg_c = _make_seg_sum(CH, ACC_T, False)


def _chunk_map():
    import numpy as np
    m = np.full((NW, CH), N_CHUNKS, np.int32)
    start = 0
    for w in range(NW):
        n = CH_BASE + (1 if w < CH_EXTRA else 0)
        m[w, :n] = np.arange(start, start + n)
        start += n
    return m


_CHUNK_MAP = _chunk_map()


def _edge_slices(src, dst, dummy):
    src = jnp.concatenate([src.reshape(N_CHUNKS, 128),
                           jnp.zeros((1, 128), jnp.int32)])
    dst = jnp.concatenate([dst.reshape(N_CHUNKS, 128),
                           jnp.full((1, 128), dummy, jnp.int32)])
    return src[_CHUNK_MAP], dst[_CHUNK_MAP]


def _proj_body(x_ref, w_ref, b_ref, o_ref):
    y = jnp.dot(x_ref[...], w_ref[0], preferred_element_type=jnp.float32)
    o_ref[...] = jnp.maximum(y + b_ref[0], 0.0)


def _input_proj(x_all, W_stack, b_stack):
    def sel(i):
        return jnp.where(i < 5, 0, jnp.where(i < 8, 1, 2))

    return pl.pallas_call(
        _proj_body,
        grid=(10,),
        in_specs=[pl.BlockSpec((1000, H), lambda i: (i, 0)),
                  pl.BlockSpec((1, H, H), lambda i: (sel(i), 0, 0)),
                  pl.BlockSpec((1, 1, H), lambda i: (sel(i), 0, 0))],
        out_specs=pl.BlockSpec((1000, H), lambda i: (i, 0)),
        out_shape=jax.ShapeDtypeStruct((10000, H), jnp.float32),
    )(x_all, W_stack, b_stack)


def _mean_from(s_ref, c_ref):
    return (s_ref[0] + s_ref[1]) / jnp.maximum(c_ref[0] + c_ref[1], 1.0)


def _ln_relu(x, g_ref, be_ref):
    mu = jnp.mean(x, axis=1, keepdims=True)
    var = jnp.mean((x - mu) * (x - mu), axis=1, keepdims=True)
    y = g_ref[...] * (x - mu) * lax.rsqrt(var + 1e-5) + be_ref[...]
    return jnp.maximum(y, 0.0)


def _update2_body(h_ref, sA_ref, cA_ref, sB_ref, cB_ref, wr_ref, wA_ref,
                  wB_ref, bc_ref, g_ref, be_ref, o_ref):
    x = jnp.dot(h_ref[...], wr_ref[...], preferred_element_type=jnp.float32)
    x = x + jnp.dot(_mean_from(sA_ref, cA_ref), wA_ref[...],
                    preferred_element_type=jnp.float32)
    x = x + jnp.dot(_mean_from(sB_ref, cB_ref), wB_ref[...],
                    preferred_element_type=jnp.float32)
    x = x + bc_ref[...]
    o_ref[...] = _ln_relu(x, g_ref, be_ref)


def _update1_body(h_ref, sA_ref, cA_ref, wr_ref, wA_ref, bc_ref, g_ref,
                  be_ref, o_ref):
    x = jnp.dot(h_ref[...], wr_ref[...], preferred_element_type=jnp.float32)
    x = x + jnp.dot(_mean_from(sA_ref, cA_ref), wA_ref[...],
                    preferred_element_type=jnp.float32)
    x = x + bc_ref[...]
    o_ref[...] = _ln_relu(x, g_ref, be_ref)


def _head_body(h_ref, sA_ref, cA_ref, wr_ref, wA_ref, bc_ref, g_ref,
               be_ref, wout_ref, bout_ref, base_ref, o_ref):
    x = jnp.dot(h_ref[...], wr_ref[...], preferred_element_type=jnp.float32)
    x = x + jnp.dot(_mean_from(sA_ref, cA_ref), wA_ref[...],
                    preferred_element_type=jnp.float32)
    x = x + bc_ref[...]
    h2 = _ln_relu(x, g_ref, be_ref)
    d = jnp.sum(h2 * wout_ref[...], axis=1, keepdims=True)
    o_ref[...] = jnp.broadcast_to(d + bout_ref[0, 0] + base_ref[0, 0],
                                  o_ref.shape)


def _row_spec(i):
    return (i, 0)


def _acc_spec(i):
    return (0, i, 0)


def _full2(shape):
    return pl.BlockSpec(shape, lambda i: (0,) * len(shape))


def _update2(h, sA, cA, sB, cB, wr, wA, wB, bc, g, be):
    n = h.shape[0]
    vec = lambda v: v.reshape(1, H)
    return pl.pallas_call(
        _update2_body,
        grid=(n // 1000,),
        in_specs=[pl.BlockSpec((1000, H), _row_spec),
                  pl.BlockSpec((2, 1000, H), _acc_spec),
                  pl.BlockSpec((2, 1000, H), _acc_spec),
                  pl.BlockSpec((2, 1000, H), _acc_spec),
                  pl.BlockSpec((2, 1000, H), _acc_spec),
                  _full2((H, H)), _full2((H, H)), _full2((H, H)),
                  _full2((1, H)), _full2((1, H)), _full2((1, H))],
        out_specs=pl.BlockSpec((1000, H), _row_spec),
        out_shape=jax.ShapeDtypeStruct((n, H), jnp.float32),
    )(h, sA, cA, sB, cB, wr, wA, wB, vec(bc), vec(g), vec(be))


def _update1(h, sA, cA, wr, wA, bc, g, be):
    n = h.shape[0]
    vec = lambda v: v.reshape(1, H)
    return pl.pallas_call(
        _update1_body,
        grid=(n // 1000,),
        in_specs=[pl.BlockSpec((1000, H), _row_spec),
                  pl.BlockSpec((2, 1000, H), _acc_spec),
                  pl.BlockSpec((2, 1000, H), _acc_spec),
                  _full2((H, H)), _full2((H, H)),
                  _full2((1, H)), _full2((1, H)), _full2((1, H))],
        out_specs=pl.BlockSpec((1000, H), _row_spec),
        out_shape=jax.ShapeDtypeStruct((n, H), jnp.float32),
    )(h, sA, cA, wr, wA, vec(bc), vec(g), vec(be))


def _head(h, sA, cA, wr, wA, bc, g, be, wout, bout, base):
    n = h.shape[0]
    vec = lambda v: v.reshape(1, H)
    full = pl.pallas_call(
        _head_body,
        grid=(n // 1000,),
        in_specs=[pl.BlockSpec((1000, H), _row_spec),
                  pl.BlockSpec((2, 1000, H), _acc_spec),
                  pl.BlockSpec((2, 1000, H), _acc_spec),
                  _full2((H, H)), _full2((H, H)),
                  _full2((1, H)), _full2((1, H)), _full2((1, H)),
                  _full2((1, H)), _full2((1, 1)), _full2((1, 1))],
        out_specs=pl.BlockSpec((1000, H), _row_spec),
        out_shape=jax.ShapeDtypeStruct((n, H), jnp.float32),
    )(h, sA, cA, wr, wA, vec(bc), vec(g), vec(be),
      wout.reshape(1, H), bout.reshape(1, 1), base.reshape(1, 1))
    return full[:, 0]


def kernel(x_papers, x_authors, x_assign, ei_r0, ei_r1, ei_r2, ei_r3,
           Win_papers, bin_papers, Win_authors, bin_authors, Win_assign,
           bin_assign, Wrel0, Wroot0, bconv0, g0, be0, Wrel1, Wroot1,
           bconv1, g1, be1, Wout, bout, base):
    x_all = jnp.concatenate([x_papers, x_authors, x_assign], axis=0)
    W_stack = jnp.stack([Win_papers, Win_authors, Win_assign])
    b_stack = jnp.stack([bin_papers, bin_authors, bin_assign]).reshape(3, 1, H)
    h0 = _input_proj(x_all, W_stack, b_stack)

    src1, dst1 = _edge_slices(ei_r1[0] + 5000, ei_r1[1], 5000)
    sums1, cnt1 = _seg_p(h0, src1, dst1)
    src3, dst3 = _edge_slices(ei_r3[0] + 8000, ei_r3[1], 5000)
    sums3, cnt3 = _seg_p(h0, src3, dst3)
    src2, dst2 = _edge_slices(ei_r2[0], ei_r2[1], 2000)
    sums2, cnt2 = _seg_t(h0, src2, dst2)

    h1p = _update2(h0[0:5000], sums1[:, 0:5000], cnt1[:, 0:5000],
                   sums3[:, 0:5000], cnt3[:, 0:5000],
                   Wroot0, Wrel0[1], Wrel0[3], bconv0, g0, be0)
    h1t = _update1(h0[8000:10000], sums2[:, 0:2000], cnt2[:, 0:2000],
                   Wroot0, Wrel0[2], bconv0, g0, be0)

    (sumsC,) = _seg_c(h1p, src2, dst2)

    return _head(h1t, sumsC[:, 0:2000], cnt2[:, 0:2000],
                 Wroot1, Wrel1[2], bconv1, g1, be1, Wout, bout, base)
